# Initial kernel scaffold; baseline (speedup 1.0000x reference)
import jax
import jax.numpy as jnp
from jax.experimental import pallas as pl


def kernel(input_labels, pos_labels, neg_labels, in_embed_ent, out_embed_ent, in_embed_rel, out_embed_rel, in_embed_map, out_embed_map):
    raise NotImplementedError("write your pallas kernel here")


# speedup vs baseline: 1.5721x; 1.5721x over previous
import functools

import jax
import jax.numpy as jnp
from jax import lax
from jax.experimental import pallas as pl
from jax.experimental.pallas import tpu as pltpu
from jax.experimental.pallas import tpu_sc as plsc

NC = 2
NS = 16
L = 16
NW = NC * NS
CB = 128


def _build_sc_gather(B, K, ENTH, D2):
    Bw = B // NW
    BKw = Bw * K
    mesh = plsc.VectorSubcoreMesh(core_axis_name="c", subcore_axis_name="s",
                                  num_cores=NC, num_subcores=NS)
    n_full, rem = divmod(BKw, 128)
    c_sizes = [128] * n_full + ([rem] if rem else [])

    @functools.partial(
        pl.kernel,
        out_type=(jax.ShapeDtypeStruct((B, D2), jnp.float32),
                  jax.ShapeDtypeStruct((B, D2), jnp.float32),
                  jax.ShapeDtypeStruct((B * K, D2), jnp.float32)),
        mesh=mesh,
        scratch_types=(
            [pltpu.VMEM((Bw,), jnp.int32),
             pltpu.VMEM((Bw,), jnp.int32),
             pltpu.VMEM((BKw,), jnp.int32),
             pltpu.VMEM((Bw,), jnp.int32),
             pltpu.VMEM((Bw,), jnp.int32)]
            + [pltpu.VMEM((n,), jnp.int32) for n in c_sizes]
            + [pltpu.VMEM((Bw, D2), jnp.float32),
               pltpu.VMEM((Bw, D2), jnp.float32)]
            + [pltpu.VMEM((n, D2), jnp.float32) for n in c_sizes]
            + [pltpu.SemaphoreType.DMA]
        ),
    )
    def sc_gather(in_lab_h, pos_lab_h, neg_lab_h, ent_in_h, ent_out_h,
                  ga_h, gb_h, gc_h, *scratch):
        nch = len(c_sizes)
        lin_v, lpos_v, lneg_v, ia_v, ib_v = scratch[:5]
        ic_vs = scratch[5:5 + nch]
        ra_v, rb_v = scratch[5 + nch:7 + nch]
        rc_vs = scratch[7 + nch:7 + 2 * nch]
        sem = scratch[7 + 2 * nch]

        wid = lax.axis_index("s") * NC + lax.axis_index("c")
        base = wid * Bw

        pltpu.sync_copy(in_lab_h.at[pl.ds(base, Bw)], lin_v)
        pltpu.sync_copy(pos_lab_h.at[pl.ds(base, Bw)], lpos_v)
        pltpu.sync_copy(neg_lab_h.at[pl.ds(base * K, BKw)], lneg_v)

        entc = jnp.int32(ENTH * 2)

        def super_idx(lab):
            o = jnp.where(lab < entc, lab, lab - entc)
            return o >> 1

        for t in range(Bw // L):
            sl = pl.ds(t * L, L)
            ia_v[sl] = super_idx(lin_v[sl])
            ib_v[sl] = super_idx(lpos_v[sl])
        for t in range(BKw // L):
            ch, off = divmod(t * L, 128)
            ic_vs[ch][pl.ds(off, L)] = super_idx(lneg_v[pl.ds(t * L, L)])

        copies = [pltpu.async_copy(ent_in_h.at[ia_v], ra_v, sem),
                  pltpu.async_copy(ent_out_h.at[ib_v], rb_v, sem)]
        for ic, rc in zip(ic_vs, rc_vs):
            copies.append(pltpu.async_copy(ent_out_h.at[ic], rc, sem))
        for c in copies:
            c.wait()

        pltpu.sync_copy(ra_v, ga_h.at[pl.ds(base, Bw), :])
        pltpu.sync_copy(rb_v, gb_h.at[pl.ds(base, Bw), :])
        off = 0
        for n, rc in zip(c_sizes, rc_vs):
            pltpu.sync_copy(rc, gc_h.at[pl.ds(base * K + off, n), :])
            off += n

    return sc_gather


def _tc_body(K, ENT, REL, D,
             il_ref, pl_ref, nl_ref, ga_ref, gb_ref, gc_ref,
             irel_ref, orel_ref, imap_ref, omap_ref, out_ref):
    f32 = jnp.float32
    entc = jnp.int32(ENT)
    rel_hi = jnp.int32(REL - 1)

    def lookup(tab_ref, idx):
        oh = (idx == lax.broadcasted_iota(jnp.int32, (idx.shape[0], REL), 1))
        return jnp.dot(oh.astype(f32), tab_ref[...], preferred_element_type=f32)

    def half_select(g, o):
        odd = (o & 1) == 1
        return jnp.where(odd, g[:, D:], g[:, :D])

    def proj(e, m):
        nrm = jnp.sqrt(jnp.sum(m * m, axis=-1, keepdims=True))
        mn = m / (nrm + 1e-8)
        return e - jnp.sum(e * mn, axis=-1, keepdims=True) * mn

    il = il_ref[...]
    lp = pl_ref[...]
    ei = il < entc
    ep = lp < entc
    oin = jnp.where(ei, il, il - entc)
    opos = jnp.where(ep, lp, lp - entc)
    oin_c = jnp.minimum(oin, rel_hi)
    opos_c = jnp.minimum(opos, rel_hi)

    a_ent = half_select(ga_ref[...], oin)
    b_ent = half_select(gb_ref[...], opos)
    a_rel = lookup(irel_ref, oin_c)
    b_rel = lookup(orel_ref, opos_c)
    m_pos = lookup(imap_ref, opos_c)
    m_in = lookup(omap_ref, oin_c)

    ei_c = ei
    ep_c = ep
    in_emb = jnp.where(ei_c, jnp.where(ep_c, a_ent, proj(a_ent, m_pos)), a_rel)
    out_emb = jnp.where(ei_c, jnp.where(ep_c, b_ent, b_rel),
                        jnp.where(ep_c, proj(b_ent, m_in), b_rel))

    acc = jax.nn.log_sigmoid(jnp.sum(in_emb * out_emb, axis=-1, keepdims=True))

    for k in range(K):
        nl = nl_ref[:, k:k + 1]
        en = nl < entc
        onk = jnp.where(en, nl, nl - entc)
        onk_c = jnp.minimum(onk, rel_hi)
        c_ent = half_select(gc_ref[:, k * 2 * D:(k + 1) * 2 * D], onk)
        c_rel = lookup(orel_ref, onk_c)
        m_neg = lookup(imap_ref, onk_c)
        en_c = en
        in_neg = jnp.where(ei_c, jnp.where(en_c, a_ent, proj(a_ent, m_neg)), a_rel)
        neg_emb = jnp.where(ei_c, jnp.where(en_c, c_ent, c_rel),
                            jnp.where(en_c, proj(c_ent, m_in), c_rel))
        acc = acc + jax.nn.log_sigmoid(
            -jnp.sum(in_neg * neg_emb, axis=-1, keepdims=True))

    out_ref[...] = -acc


def kernel(input_labels, pos_labels, neg_labels, in_embed_ent, out_embed_ent,
           in_embed_rel, out_embed_rel, in_embed_map, out_embed_map):
    B = input_labels.shape[0]
    K = neg_labels.shape[1]
    ENT, D = in_embed_ent.shape
    REL = in_embed_rel.shape[0]

    il = input_labels.astype(jnp.int32)
    lp = pos_labels.astype(jnp.int32)
    nl = neg_labels.astype(jnp.int32)

    ent2_in = in_embed_ent.reshape(ENT // 2, 2 * D)
    ent2_out = out_embed_ent.reshape(ENT // 2, 2 * D)

    sc_gather = _build_sc_gather(B, K, ENT // 2, 2 * D)
    ga, gb, gc = sc_gather(il, lp, nl.reshape(B * K), ent2_in, ent2_out)
    gc2 = gc.reshape(B, K * 2 * D)

    nl_pad = jnp.zeros((B, 128), jnp.int32).at[:, :K].set(nl)

    grid = (B // CB,)
    body = functools.partial(_tc_body, K, ENT, REL, D)
    tbl_spec = pl.BlockSpec((REL, D), lambda i: (0, 0))
    out = pl.pallas_call(
        body,
        grid=grid,
        in_specs=[
            pl.BlockSpec((CB, 1), lambda i: (i, 0)),
            pl.BlockSpec((CB, 1), lambda i: (i, 0)),
            pl.BlockSpec((CB, 128), lambda i: (i, 0)),
            pl.BlockSpec((CB, 2 * D), lambda i: (i, 0)),
            pl.BlockSpec((CB, 2 * D), lambda i: (i, 0)),
            pl.BlockSpec((CB, K * 2 * D), lambda i: (i, 0)),
            tbl_spec, tbl_spec, tbl_spec, tbl_spec,
        ],
        out_specs=pl.BlockSpec((CB, 1), lambda i: (i, 0)),
        out_shape=jax.ShapeDtypeStruct((B, 1), jnp.float32),
    )(il.reshape(B, 1), lp.reshape(B, 1), nl_pad, ga, gb, gc2,
      in_embed_rel, out_embed_rel, in_embed_map, out_embed_map)
    return out.reshape(B)


# speedup vs baseline: 1.9504x; 1.2406x over previous
import functools

import jax
import jax.numpy as jnp
from jax import lax
from jax.experimental import pallas as pl
from jax.experimental.pallas import tpu as pltpu
from jax.experimental.pallas import tpu_sc as plsc

NC = 2
NS = 16
L = 16
NW = NC * NS
CB = 128


def _build_sc_gather(B, K, ENT, D2):
    Bw = B // NW
    BKw = Bw * K
    mesh = plsc.VectorSubcoreMesh(core_axis_name="c", subcore_axis_name="s",
                                  num_cores=NC, num_subcores=NS)
    n_full, rem = divmod(BKw, 128)
    c_sizes = [128] * n_full + ([rem] if rem else [])

    @functools.partial(
        pl.kernel,
        out_type=(jax.ShapeDtypeStruct((B, D2), jnp.float32),
                  jax.ShapeDtypeStruct((B, D2), jnp.float32),
                  jax.ShapeDtypeStruct((B * K, D2), jnp.float32)),
        mesh=mesh,
        scratch_types=(
            [pltpu.VMEM((Bw,), jnp.int32),
             pltpu.VMEM((Bw,), jnp.int32),
             pltpu.VMEM((BKw,), jnp.int32),
             pltpu.VMEM((Bw,), jnp.int32),
             pltpu.VMEM((Bw,), jnp.int32)]
            + [pltpu.VMEM((n,), jnp.int32) for n in c_sizes]
            + [pltpu.VMEM((Bw, D2), jnp.float32),
               pltpu.VMEM((Bw, D2), jnp.float32)]
            + [pltpu.VMEM((n, D2), jnp.float32) for n in c_sizes]
            + [pltpu.SemaphoreType.DMA]
        ),
        compiler_params=pltpu.CompilerParams(use_tc_tiling_on_sc=False),
    )
    def sc_gather(in_lab_h, pos_lab_h, neg_lab_h, ent_in_h, ent_out_h,
                  ga_h, gb_h, gc_h, *scratch):
        nch = len(c_sizes)
        lin_v, lpos_v, lneg_v, ia_v, ib_v = scratch[:5]
        ic_vs = scratch[5:5 + nch]
        ra_v, rb_v = scratch[5 + nch:7 + nch]
        rc_vs = scratch[7 + nch:7 + 2 * nch]
        sem = scratch[7 + 2 * nch]

        wid = lax.axis_index("s") * NC + lax.axis_index("c")
        base = wid * Bw

        pltpu.sync_copy(in_lab_h.at[pl.ds(base, Bw)], lin_v)
        pltpu.sync_copy(pos_lab_h.at[pl.ds(base, Bw)], lpos_v)
        pltpu.sync_copy(neg_lab_h.at[pl.ds(base * K, BKw)], lneg_v)

        entc = jnp.int32(ENT)

        def super_idx(lab):
            return jnp.where(lab < entc, lab, lab - entc)

        for t in range(Bw // L):
            sl = pl.ds(t * L, L)
            ia_v[sl] = super_idx(lin_v[sl])
            ib_v[sl] = super_idx(lpos_v[sl])
        for t in range(BKw // L):
            ch, off = divmod(t * L, 128)
            ic_vs[ch][pl.ds(off, L)] = super_idx(lneg_v[pl.ds(t * L, L)])

        copies = [pltpu.async_copy(ent_in_h.at[ia_v], ra_v, sem),
                  pltpu.async_copy(ent_out_h.at[ib_v], rb_v, sem)]
        for ic, rc in zip(ic_vs, rc_vs):
            copies.append(pltpu.async_copy(ent_out_h.at[ic], rc, sem))
        for c in copies:
            c.wait()

        pltpu.sync_copy(ra_v, ga_h.at[pl.ds(base, Bw), :])
        pltpu.sync_copy(rb_v, gb_h.at[pl.ds(base, Bw), :])
        off = 0
        for n, rc in zip(c_sizes, rc_vs):
            pltpu.sync_copy(rc, gc_h.at[pl.ds(base * K + off, n), :])
            off += n

    return sc_gather


def _tc_body(K, ENT, REL, D,
             il_ref, pl_ref, nl_ref, ga_ref, gb_ref, gc_ref,
             irel_ref, orel_ref, imap_ref, omap_ref, out_ref):
    f32 = jnp.float32
    entc = jnp.int32(ENT)
    rel_hi = jnp.int32(REL - 1)

    def lookup(tab_ref, idx):
        oh = (idx == lax.broadcasted_iota(jnp.int32, (idx.shape[0], REL), 1))
        return jnp.dot(oh.astype(f32), tab_ref[...], preferred_element_type=f32)

    def proj(e, m):
        nrm = jnp.sqrt(jnp.sum(m * m, axis=-1, keepdims=True))
        mn = m / (nrm + 1e-8)
        return e - jnp.sum(e * mn, axis=-1, keepdims=True) * mn

    il = il_ref[...]
    lp = pl_ref[...]
    ei = il < entc
    ep = lp < entc
    oin = jnp.where(ei, il, il - entc)
    opos = jnp.where(ep, lp, lp - entc)
    oin_c = jnp.minimum(oin, rel_hi)
    opos_c = jnp.minimum(opos, rel_hi)

    a_ent = ga_ref[...]
    b_ent = gb_ref[...]
    a_rel = lookup(irel_ref, oin_c)
    b_rel = lookup(orel_ref, opos_c)
    m_pos = lookup(imap_ref, opos_c)
    m_in = lookup(omap_ref, oin_c)

    ei_c = ei
    ep_c = ep
    in_emb = jnp.where(ei_c, jnp.where(ep_c, a_ent, proj(a_ent, m_pos)), a_rel)
    out_emb = jnp.where(ei_c, jnp.where(ep_c, b_ent, b_rel),
                        jnp.where(ep_c, proj(b_ent, m_in), b_rel))

    acc = jax.nn.log_sigmoid(jnp.sum(in_emb * out_emb, axis=-1, keepdims=True))

    gc3 = gc_ref[...].reshape(il.shape[0], K, D)
    for k in range(K):
        nl = nl_ref[:, k:k + 1]
        en = nl < entc
        onk = jnp.where(en, nl, nl - entc)
        onk_c = jnp.minimum(onk, rel_hi)
        c_ent = gc3[:, k, :]
        c_rel = lookup(orel_ref, onk_c)
        m_neg = lookup(imap_ref, onk_c)
        en_c = en
        in_neg = jnp.where(ei_c, jnp.where(en_c, a_ent, proj(a_ent, m_neg)), a_rel)
        neg_emb = jnp.where(ei_c, jnp.where(en_c, c_ent, c_rel),
                            jnp.where(en_c, proj(c_ent, m_in), c_rel))
        acc = acc + jax.nn.log_sigmoid(
            -jnp.sum(in_neg * neg_emb, axis=-1, keepdims=True))

    out_ref[...] = -acc


def kernel(input_labels, pos_labels, neg_labels, in_embed_ent, out_embed_ent,
           in_embed_rel, out_embed_rel, in_embed_map, out_embed_map):
    B = input_labels.shape[0]
    K = neg_labels.shape[1]
    ENT, D = in_embed_ent.shape
    REL = in_embed_rel.shape[0]

    il = input_labels.astype(jnp.int32)
    lp = pos_labels.astype(jnp.int32)
    nl = neg_labels.astype(jnp.int32)

    sc_gather = _build_sc_gather(B, K, ENT, D)
    ga, gb, gc = sc_gather(il, lp, nl.reshape(B * K), in_embed_ent, out_embed_ent)

    nl_pad = jnp.zeros((B, 128), jnp.int32).at[:, :K].set(nl)

    grid = (B // CB,)
    body = functools.partial(_tc_body, K, ENT, REL, D)
    tbl_spec = pl.BlockSpec((REL, D), lambda i: (0, 0))
    out = pl.pallas_call(
        body,
        grid=grid,
        in_specs=[
            pl.BlockSpec((CB, 1), lambda i: (i, 0)),
            pl.BlockSpec((CB, 1), lambda i: (i, 0)),
            pl.BlockSpec((CB, 128), lambda i: (i, 0)),
            pl.BlockSpec((CB, D), lambda i: (i, 0)),
            pl.BlockSpec((CB, D), lambda i: (i, 0)),
            pl.BlockSpec((CB * K, D), lambda i: (i, 0)),
            tbl_spec, tbl_spec, tbl_spec, tbl_spec,
        ],
        out_specs=pl.BlockSpec((CB, 1), lambda i: (i, 0)),
        out_shape=jax.ShapeDtypeStruct((B, 1), jnp.float32),
    )(il.reshape(B, 1), lp.reshape(B, 1), nl_pad, ga, gb, gc,
      in_embed_rel, out_embed_rel, in_embed_map, out_embed_map)
    return out.reshape(B)
